# Initial kernel scaffold; baseline (speedup 1.0000x reference)
import jax
import jax.numpy as jnp
from jax.experimental import pallas as pl


def kernel(features, edge_index, W1, b1, W3, b3):
    raise NotImplementedError("write your pallas kernel here")


# speedup vs baseline: 6.2902x; 6.2902x over previous
import functools

import jax
import jax.numpy as jnp
from jax import lax
from jax.experimental import pallas as pl
from jax.experimental.pallas import tpu as pltpu
from jax.experimental.pallas import tpu_sc as plsc

_N = 10000
_E = 320000
_D = 128
_K = 4
_NPAD = 10240
_EC = 80
_NCORE = 2
_NSUB = 16
_NW = _NCORE * _NSUB
_EPW = _E // _NW
_NCH = _EPW // _EC
_EPS = _E // _NSUB
_DCH = _EPS // _EC
_RPS = _NPAD // _NSUB
_DPS = _NPAD // _NSUB
_RB = 2000


def _sc_mesh():
    return plsc.VectorSubcoreMesh(core_axis_name="c", subcore_axis_name="s")


def _degrees(edges4, zeros_d8, ones8):
    @functools.partial(
        pl.kernel,
        out_type=jax.ShapeDtypeStruct((_NCORE, _NPAD, _D), jnp.float32),
        mesh=_sc_mesh(),
        scratch_types=[
            pltpu.VMEM((_DCH, _EC), jnp.int32),
            pltpu.VMEM((_EC, _D), jnp.float32),
            pltpu.VMEM_SHARED((_NPAD, _D), jnp.float32),
        ],
    )
    def deg_kernel(e_hbm, z_hbm, ones_hbm, out_hbm, idxv, onesv, deg_sp):
        c = lax.axis_index("c")
        s = lax.axis_index("s")
        pltpu.sync_copy(z_hbm.at[pl.ds(s * _DPS, _DPS)],
                        deg_sp.at[pl.ds(s * _DPS, _DPS)])
        pltpu.sync_copy(e_hbm.at[c, s], idxv)
        pltpu.sync_copy(ones_hbm, onesv)
        plsc.subcore_barrier()

        def chunk(i, carry):
            pltpu.sync_copy(onesv, deg_sp.at[idxv.at[i]], add=True)
            return carry

        lax.fori_loop(0, _DCH, chunk, 0)
        plsc.subcore_barrier()
        pltpu.sync_copy(deg_sp.at[pl.ds(s * _DPS, _DPS)],
                        out_hbm.at[c, pl.ds(s * _DPS, _DPS)])

    return deg_kernel(edges4, zeros_d8, ones8)


def _prop(g, src3, dst3, zeros_nd):
    @functools.partial(
        pl.kernel,
        out_type=jax.ShapeDtypeStruct((_NCORE, _NPAD, _D), jnp.float32),
        mesh=_sc_mesh(),
        scratch_types=[
            pltpu.VMEM((_NCH, _EC), jnp.int32),
            pltpu.VMEM((_NCH, _EC), jnp.int32),
            pltpu.VMEM((_EC, _D), jnp.float32),
            pltpu.VMEM_SHARED((_NPAD, _D), jnp.float32),
        ],
    )
    def prop_kernel(g_hbm, src_hbm, dst_hbm, z_hbm, out_hbm,
                    srcv, dstv, rows, agg_sp):
        c = lax.axis_index("c")
        s = lax.axis_index("s")
        w = c * _NSUB + s
        pltpu.sync_copy(z_hbm.at[pl.ds(s * _RPS, _RPS)],
                        agg_sp.at[pl.ds(s * _RPS, _RPS)])
        pltpu.sync_copy(src_hbm.at[w], srcv)
        pltpu.sync_copy(dst_hbm.at[w], dstv)
        plsc.subcore_barrier()

        def chunk(i, carry):
            pltpu.sync_copy(g_hbm.at[srcv.at[i]], rows)
            pltpu.sync_copy(rows, agg_sp.at[dstv.at[i]], add=True)
            return carry

        lax.fori_loop(0, _NCH, chunk, 0)
        plsc.subcore_barrier()
        pltpu.sync_copy(agg_sp.at[pl.ds(s * _RPS, _RPS)],
                        out_hbm.at[c, pl.ds(s * _RPS, _RPS)])

    return prop_kernel(g, src3, dst3, zeros_nd)


def _norm_col(d_ref):
    d = d_ref[:, 0:1]
    return jnp.where(d > 0.0, lax.rsqrt(d), 0.0)


def _row_spec():
    return pl.BlockSpec((_RB, _D), lambda i: (i, 0))


def _deg_spec():
    return pl.BlockSpec((_RB, _D), lambda i: (i, 0))


def _agg_spec():
    return pl.BlockSpec((_NCORE, _RB, _D), lambda i: (0, i, 0))


def _mat_spec():
    return pl.BlockSpec((_D, _D), lambda i: (0, 0))


def _bias_spec():
    return pl.BlockSpec((1, _D), lambda i: (0, 0))


_OUT_ND = jax.ShapeDtypeStruct((_N, _D), jnp.float32)
_GRID = (_N // _RB,)


def _mm1(x, w1, dq_out):
    def body(x_ref, w_ref, do_ref, o_ref):
        no = _norm_col(do_ref)
        o_ref[...] = jnp.dot(x_ref[...], w_ref[...],
                             preferred_element_type=jnp.float32) * no

    return pl.pallas_call(
        body, grid=_GRID,
        in_specs=[_row_spec(), _mat_spec(), _deg_spec()],
        out_specs=_row_spec(), out_shape=_OUT_ND,
    )(x, w1, dq_out)


def _hop(agg, dq_out, dq_in, bias):
    with_bias = bias is not None

    def body(a_ref, do_ref, di_ref, *rest):
        o_ref = rest[-1]
        a = a_ref[0] + a_ref[1]
        no = _norm_col(do_ref)
        ni = _norm_col(di_ref)
        g = a * (no * ni)
        if with_bias:
            g = g + no * rest[0][...]
        o_ref[...] = g

    in_specs = [_agg_spec(), _deg_spec(), _deg_spec()]
    args = [agg, dq_out, dq_in]
    if with_bias:
        in_specs.append(_bias_spec())
        args.append(bias)
    return pl.pallas_call(
        body, grid=_GRID, in_specs=in_specs,
        out_specs=_row_spec(), out_shape=_OUT_ND,
    )(*args)


def _mm2(agg, dq_in, dq_out, w3):
    def body(a_ref, di_ref, do_ref, w_ref, o_ref):
        a = a_ref[0] + a_ref[1]
        h = a * _norm_col(di_ref)
        o_ref[...] = jnp.dot(h, w_ref[...],
                             preferred_element_type=jnp.float32) * _norm_col(do_ref)

    return pl.pallas_call(
        body, grid=_GRID,
        in_specs=[_agg_spec(), _deg_spec(), _deg_spec(), _mat_spec()],
        out_specs=_row_spec(), out_shape=_OUT_ND,
    )(agg, dq_in, dq_out, w3)


def _final(agg, dq_in, bias):
    def body(a_ref, di_ref, b_ref, o_ref):
        a = a_ref[0] + a_ref[1]
        o_ref[...] = a * _norm_col(di_ref) + b_ref[...]

    return pl.pallas_call(
        body, grid=_GRID,
        in_specs=[_agg_spec(), _deg_spec(), _bias_spec()],
        out_specs=_row_spec(), out_shape=_OUT_ND,
    )(agg, dq_in, bias)


def kernel(features, edge_index, W1, b1, W3, b3):
    edges4 = edge_index.reshape(2, _NSUB, _DCH, _EC)
    src3 = edge_index[0].reshape(_NW, _NCH, _EC)
    dst3 = edge_index[1].reshape(_NW, _NCH, _EC)
    zeros_nd = jnp.zeros((_NPAD, _D), jnp.float32)
    ones_d = jnp.ones((_EC, _D), jnp.float32)

    deg = _degrees(edges4, zeros_nd, ones_d)
    dq_out = deg[0, :_N]
    dq_in = deg[1, :_N]

    b1r = b1.reshape(1, _D)
    b3r = b3.reshape(1, _D)

    g = _mm1(features, W1, dq_out)
    a = _prop(g, src3, dst3, zeros_nd)
    g = _hop(a, dq_out, dq_in, b1r)
    for _ in range(_K - 1):
        a = _prop(g, src3, dst3, zeros_nd)
        g = _hop(a, dq_out, dq_in, None)
    a = _prop(g, src3, dst3, zeros_nd)
    g = _mm2(a, dq_in, dq_out, W3)
    a = _prop(g, src3, dst3, zeros_nd)
    return _final(a, dq_in, b3r)


# speedup vs baseline: 10.2835x; 1.6348x over previous
import functools

import jax
import jax.numpy as jnp
from jax import lax
from jax.experimental import pallas as pl
from jax.experimental.pallas import tpu as pltpu
from jax.experimental.pallas import tpu_sc as plsc

_N = 10000
_E = 320000
_D = 128
_K = 4
_NPAD = 10240
_EC = 128
_ECD = 80
_NCORE = 2
_NSUB = 16
_NW = _NCORE * _NSUB
_EPAD = _NW * 80 * _EC
_EPW = _EPAD // _NW
_NCH = _EPW // _EC
_NST = 2
_SCH = _NCH // _NST
_EPS = _E // _NSUB
_DCH = _EPS // _ECD
_RPS = _NPAD // _NSUB
_DPS = _NPAD // _NSUB
_RB = 2048


def _sc_mesh():
    return plsc.VectorSubcoreMesh(core_axis_name="c", subcore_axis_name="s")


def _degrees(edges4, zeros_d8, ones8):
    @functools.partial(
        pl.kernel,
        out_type=jax.ShapeDtypeStruct((_NCORE, _NPAD, _D), jnp.float32),
        mesh=_sc_mesh(),
        scratch_types=[
            pltpu.VMEM((_DCH, _ECD), jnp.int32),
            pltpu.VMEM((_ECD, _D), jnp.float32),
            pltpu.VMEM_SHARED((_NPAD, _D), jnp.float32),
            pltpu.SemaphoreType.DMA,
            pltpu.SemaphoreType.DMA,
        ],
    )
    def deg_kernel(e_hbm, z_hbm, ones_hbm, out_hbm, idxv, onesv, deg_sp,
                   sem_a, sem_b):
        c = lax.axis_index("c")
        s = lax.axis_index("s")
        pltpu.sync_copy(z_hbm.at[pl.ds(s * _DPS, _DPS)],
                        deg_sp.at[pl.ds(s * _DPS, _DPS)])
        pltpu.sync_copy(e_hbm.at[c, s], idxv)
        pltpu.sync_copy(ones_hbm, onesv)
        plsc.subcore_barrier()

        def pair(j, carry):
            @pl.when(j >= 1)
            def _():
                pltpu.make_async_copy(
                    onesv, deg_sp.at[idxv.at[2 * j - 2]], sem_a).wait()

            pltpu.async_copy(onesv, deg_sp.at[idxv.at[2 * j]],
                             sem_a, add=True)

            @pl.when(j >= 1)
            def _():
                pltpu.make_async_copy(
                    onesv, deg_sp.at[idxv.at[2 * j - 1]], sem_b).wait()

            pltpu.async_copy(onesv, deg_sp.at[idxv.at[2 * j + 1]],
                             sem_b, add=True)
            return carry

        lax.fori_loop(0, _DCH // 2, pair, 0)
        pltpu.make_async_copy(onesv, deg_sp.at[idxv.at[_DCH - 2]],
                              sem_a).wait()
        pltpu.make_async_copy(onesv, deg_sp.at[idxv.at[_DCH - 1]],
                              sem_b).wait()
        plsc.subcore_barrier()
        pltpu.sync_copy(deg_sp.at[pl.ds(s * _DPS, _DPS)],
                        out_hbm.at[c, pl.ds(s * _DPS, _DPS)])

    return deg_kernel(edges4, zeros_d8, ones8)


def _prop(g, src3, dst3, zeros_nd):
    @functools.partial(
        pl.kernel,
        out_type=jax.ShapeDtypeStruct((_NCORE, _NPAD, _D), jnp.float32),
        mesh=_sc_mesh(),
        scratch_types=[
            pltpu.VMEM((_SCH, _EC), jnp.int32),
            pltpu.VMEM((_SCH, _EC), jnp.int32),
            pltpu.VMEM((_EC, _D), jnp.float32),
            pltpu.VMEM((_EC, _D), jnp.float32),
            pltpu.VMEM_SHARED((_NPAD, _D), jnp.float32),
            pltpu.SemaphoreType.DMA,
            pltpu.SemaphoreType.DMA,
            pltpu.SemaphoreType.DMA,
            pltpu.SemaphoreType.DMA,
        ],
    )
    def prop_kernel(g_hbm, src_hbm, dst_hbm, z_hbm, out_hbm,
                    srcv, dstv, rows_a, rows_b, agg_sp,
                    gsem_a, gsem_b, ssem_a, ssem_b):
        c = lax.axis_index("c")
        s = lax.axis_index("s")
        w = c * _NSUB + s
        pltpu.sync_copy(z_hbm.at[pl.ds(s * _RPS, _RPS)],
                        agg_sp.at[pl.ds(s * _RPS, _RPS)])
        plsc.subcore_barrier()

        for stage in range(_NST):
            base = stage * _SCH
            pltpu.sync_copy(src_hbm.at[w, pl.ds(base, _SCH)], srcv)
            pltpu.sync_copy(dst_hbm.at[w, pl.ds(base, _SCH)], dstv)
            pltpu.async_copy(g_hbm.at[srcv.at[0]], rows_a, gsem_a)

            def pair(j, carry):
                @pl.when(j >= 1)
                def _():
                    pltpu.make_async_copy(rows_b,
                                          agg_sp.at[dstv.at[2 * j - 1]],
                                          ssem_b).wait()

                pltpu.async_copy(g_hbm.at[srcv.at[2 * j + 1]], rows_b, gsem_b)
                pltpu.make_async_copy(g_hbm.at[srcv.at[2 * j]],
                                      rows_a, gsem_a).wait()
                pltpu.async_copy(rows_a, agg_sp.at[dstv.at[2 * j]],
                                 ssem_a, add=True)

                pltpu.make_async_copy(rows_a, agg_sp.at[dstv.at[2 * j]],
                                      ssem_a).wait()

                @pl.when(j + 1 < _SCH // 2)
                def _():
                    pltpu.async_copy(g_hbm.at[srcv.at[2 * j + 2]],
                                     rows_a, gsem_a)

                pltpu.make_async_copy(g_hbm.at[srcv.at[2 * j + 1]],
                                      rows_b, gsem_b).wait()
                pltpu.async_copy(rows_b, agg_sp.at[dstv.at[2 * j + 1]],
                                 ssem_b, add=True)
                return carry

            lax.fori_loop(0, _SCH // 2, pair, 0)
            pltpu.make_async_copy(rows_b, agg_sp.at[dstv.at[_SCH - 1]],
                                  ssem_b).wait()
        plsc.subcore_barrier()
        pltpu.sync_copy(agg_sp.at[pl.ds(s * _RPS, _RPS)],
                        out_hbm.at[c, pl.ds(s * _RPS, _RPS)])

    return prop_kernel(g, src3, dst3, zeros_nd)


def _norm_col(d_ref):
    d = d_ref[:, 0:1]
    return jnp.where(d > 0.0, lax.rsqrt(d), 0.0)


def _row_spec():
    return pl.BlockSpec((_RB, _D), lambda i: (i, 0))


def _deg_spec():
    return pl.BlockSpec((_RB, _D), lambda i: (i, 0))


def _agg_spec():
    return pl.BlockSpec((_NCORE, _RB, _D), lambda i: (0, i, 0))


def _mat_spec():
    return pl.BlockSpec((_D, _D), lambda i: (0, 0))


def _bias_spec():
    return pl.BlockSpec((1, _D), lambda i: (0, 0))


_OUT_ND = jax.ShapeDtypeStruct((_NPAD, _D), jnp.float32)
_GRID = (_NPAD // _RB,)


def _mm1(x, w1, dq_out):
    def body(x_ref, w_ref, do_ref, o_ref):
        no = _norm_col(do_ref)
        o_ref[...] = jnp.dot(x_ref[...], w_ref[...],
                             preferred_element_type=jnp.float32) * no

    return pl.pallas_call(
        body, grid=_GRID,
        in_specs=[_row_spec(), _mat_spec(), _deg_spec()],
        out_specs=_row_spec(), out_shape=_OUT_ND,
    )(x, w1, dq_out)


def _hop(agg, dq_out, dq_in, bias):
    with_bias = bias is not None

    def body(a_ref, do_ref, di_ref, *rest):
        o_ref = rest[-1]
        a = a_ref[0] + a_ref[1]
        no = _norm_col(do_ref)
        ni = _norm_col(di_ref)
        g = a * (no * ni)
        if with_bias:
            g = g + no * rest[0][...]
        o_ref[...] = g

    in_specs = [_agg_spec(), _deg_spec(), _deg_spec()]
    args = [agg, dq_out, dq_in]
    if with_bias:
        in_specs.append(_bias_spec())
        args.append(bias)
    return pl.pallas_call(
        body, grid=_GRID, in_specs=in_specs,
        out_specs=_row_spec(), out_shape=_OUT_ND,
    )(*args)


def _mm2(agg, dq_in, dq_out, w3):
    def body(a_ref, di_ref, do_ref, w_ref, o_ref):
        a = a_ref[0] + a_ref[1]
        h = a * _norm_col(di_ref)
        o_ref[...] = jnp.dot(h, w_ref[...],
                             preferred_element_type=jnp.float32) * _norm_col(do_ref)

    return pl.pallas_call(
        body, grid=_GRID,
        in_specs=[_agg_spec(), _deg_spec(), _deg_spec(), _mat_spec()],
        out_specs=_row_spec(), out_shape=_OUT_ND,
    )(agg, dq_in, dq_out, w3)


def _final(agg, dq_in, bias):
    def body(a_ref, di_ref, b_ref, o_ref):
        a = a_ref[0] + a_ref[1]
        o_ref[...] = a * _norm_col(di_ref) + b_ref[...]

    return pl.pallas_call(
        body, grid=_GRID,
        in_specs=[_agg_spec(), _deg_spec(), _bias_spec()],
        out_specs=_row_spec(), out_shape=_OUT_ND,
    )(agg, dq_in, bias)


def kernel(features, edge_index, W1, b1, W3, b3):
    sink = (_N + (jnp.arange(_EPAD - _E, dtype=jnp.int32)
                  % (_NPAD - _N))).astype(jnp.int32)
    src3 = jnp.concatenate([edge_index[0], sink]).reshape(_NW, _NCH, _EC)
    dst3 = jnp.concatenate([edge_index[1], sink]).reshape(_NW, _NCH, _EC)
    edges4 = edge_index.reshape(2, _NSUB, _DCH, _ECD)
    zeros_nd = jnp.zeros((_NPAD, _D), jnp.float32)
    ones_d = jnp.ones((_ECD, _D), jnp.float32)
    xp = jnp.concatenate(
        [features, jnp.zeros((_NPAD - _N, _D), jnp.float32)])

    deg = _degrees(edges4, zeros_nd, ones_d)
    dq_out = deg[0]
    dq_in = deg[1]

    b1r = b1.reshape(1, _D)
    b3r = b3.reshape(1, _D)

    g = _mm1(xp, W1, dq_out)
    a = _prop(g, src3, dst3, zeros_nd)
    g = _hop(a, dq_out, dq_in, b1r)
    for _ in range(_K - 1):
        a = _prop(g, src3, dst3, zeros_nd)
        g = _hop(a, dq_out, dq_in, None)
    a = _prop(g, src3, dst3, zeros_nd)
    g = _mm2(a, dq_in, dq_out, W3)
    a = _prop(g, src3, dst3, zeros_nd)
    return _final(a, dq_in, b3r)[:_N]
